# baseline (speedup 1.0000x reference)
import functools

import jax
import jax.numpy as jnp
from jax import lax
from jax.experimental import pallas as pl
from jax.experimental.pallas import tpu as pltpu
from jax.experimental.pallas import tpu_sc as plsc

K = 8192
D = 64
N = 16384
BETA = 0.25

TM = 512
TK = 512


def _project_body(cb_ref, pw_ref, pb_ref, qc_ref, qcsq_ref):
    qc = lax.dot_general(
        cb_ref[...], pw_ref[...],
        dimension_numbers=(((1,), (1,)), ((), ())),
        preferred_element_type=jnp.float32,
        precision=lax.Precision.DEFAULT,
    ) + pb_ref[...]
    qc_ref[...] = qc
    qcsq_ref[...] = jnp.sum(qc * qc, axis=1)


def _project(codebook, proj_w, proj_b):
    return pl.pallas_call(
        _project_body,
        out_shape=(
            jax.ShapeDtypeStruct((K, D), jnp.float32),
            jax.ShapeDtypeStruct((K,), jnp.float32),
        ),
    )(codebook, proj_w, proj_b.reshape(1, D))


def _argmin_body(z_ref, qc_ref, qcsq_ref, out_ref, rmin_ref, rarg_ref):
    j = pl.program_id(1)
    nj = pl.num_programs(1)
    zb = z_ref[...]
    zsq = jnp.sum(zb * zb, axis=1)
    s = lax.dot_general(
        zb, qc_ref[...],
        dimension_numbers=(((1,), (1,)), ((), ())),
        preferred_element_type=jnp.float32,
        precision=lax.Precision.DEFAULT,
    )
    d = (zsq[:, None] + qcsq_ref[...][None, :]) - 2.0 * s
    m = jnp.min(d, axis=1)
    iota = lax.broadcasted_iota(jnp.int32, (TM, TK), 1)
    a = jnp.min(jnp.where(d == m[:, None], iota, jnp.int32(2**30)), axis=1)
    a = a + j * TK

    @pl.when(j == 0)
    def _():
        rmin_ref[...] = m
        rarg_ref[...] = a

    @pl.when(j > 0)
    def _():
        better = m < rmin_ref[...]
        rarg_ref[...] = jnp.where(better, a, rarg_ref[...])
        rmin_ref[...] = jnp.minimum(m, rmin_ref[...])

    @pl.when(j == nj - 1)
    def _():
        out_ref[...] = rarg_ref[...]


def _argmin(zf, qc, qcsq):
    return pl.pallas_call(
        _argmin_body,
        grid=(N // TM, K // TK),
        in_specs=[
            pl.BlockSpec((TM, D), lambda i, j: (i, 0)),
            pl.BlockSpec((TK, D), lambda i, j: (j, 0)),
            pl.BlockSpec((TK,), lambda i, j: (j,)),
        ],
        out_specs=pl.BlockSpec((TM,), lambda i, j: (i,)),
        out_shape=jax.ShapeDtypeStruct((N,), jnp.int32),
        scratch_shapes=[
            pltpu.VMEM((TM,), jnp.float32),
            pltpu.VMEM((TM,), jnp.int32),
        ],
        compiler_params=pltpu.CompilerParams(
            dimension_semantics=("arbitrary", "arbitrary"),
        ),
    )(zf, qc, qcsq)


_SC_NC = 2
_SC_NS = 16
_BPW = N // (_SC_NC * _SC_NS)
_KPS = K // _SC_NS


def _sc_body(qc_hbm, idx_hbm, zq_hbm, cnt_hbm,
             idx_v, rows_v, stage_v, ones_v, cnt_sh, sem):
    c = lax.axis_index("c")
    s = lax.axis_index("s")
    wid = c * _SC_NS + s
    base = wid * _BPW

    pltpu.sync_copy(idx_hbm.at[pl.ds(base, _BPW)], idx_v)
    pltpu.async_copy(qc_hbm.at[idx_v], rows_v, sem).wait()
    pltpu.sync_copy(rows_v, zq_hbm.at[pl.ds(base, _BPW)])

    def fill(i, _):
        ones_v[pl.ds(i * 16, 16)] = jnp.full((16,), 1, jnp.int32)
        stage_v[pl.ds(i * 16, 16)] = jnp.full((16,), 0, jnp.int32)
        return 0

    lax.fori_loop(0, _BPW // 16, fill, 0)

    pltpu.sync_copy(stage_v, cnt_sh.at[pl.ds(s * _KPS, _KPS)])
    plsc.subcore_barrier()
    pltpu.sync_copy(ones_v, cnt_sh.at[idx_v], add=True)
    plsc.subcore_barrier()

    pltpu.sync_copy(cnt_sh.at[pl.ds(s * _KPS, _KPS)], stage_v)
    pltpu.sync_copy(stage_v, cnt_hbm.at[c, pl.ds(s * _KPS, _KPS)])


def _gather_counts(qc, idx):
    mesh = plsc.VectorSubcoreMesh(core_axis_name="c", subcore_axis_name="s")
    f = pl.kernel(
        _sc_body,
        out_type=(
            jax.ShapeDtypeStruct((N, D), jnp.float32),
            jax.ShapeDtypeStruct((_SC_NC, K), jnp.int32),
        ),
        mesh=mesh,
        scratch_types=[
            pltpu.VMEM((_BPW,), jnp.int32),
            pltpu.VMEM((_BPW, D), jnp.float32),
            pltpu.VMEM((_KPS,), jnp.int32),
            pltpu.VMEM((_BPW,), jnp.int32),
            pltpu.VMEM_SHARED((K,), jnp.int32),
            pltpu.SemaphoreType.DMA,
        ],
        compiler_params=pltpu.CompilerParams(use_tc_tiling_on_sc=False),
    )
    return f(qc, idx)


def _loss_body(zf_ref, zq_ref, cnt_ref, loss_ref, perp_ref):
    diff = zq_ref[...] - zf_ref[...]
    sq = jnp.sum(diff * diff)
    loss_ref[0, 0] = (1.0 + BETA) * sq / jnp.float32(N * D)
    counts = cnt_ref[0:K] + cnt_ref[K:2 * K]
    e = counts.astype(jnp.float32) * jnp.float32(1.0 / N)
    ent = jnp.sum(e * jnp.log(e + 1e-8))
    perp_ref[0, 0] = jnp.exp(-ent)


def _losses(zf, zq, cnt):
    return pl.pallas_call(
        _loss_body,
        in_specs=[
            pl.BlockSpec(memory_space=pltpu.VMEM),
            pl.BlockSpec(memory_space=pltpu.VMEM),
            pl.BlockSpec(memory_space=pltpu.VMEM),
        ],
        out_specs=(
            pl.BlockSpec(memory_space=pltpu.SMEM),
            pl.BlockSpec(memory_space=pltpu.SMEM),
        ),
        out_shape=(
            jax.ShapeDtypeStruct((1, 1), jnp.float32),
            jax.ShapeDtypeStruct((1, 1), jnp.float32),
        ),
    )(zf, zq, cnt)


def kernel(z, codebook, proj_w, proj_b):
    zf = z.reshape(-1, D)
    qc, qcsq = _project(codebook, proj_w, proj_b)
    idx = _argmin(zf, qc, qcsq)
    zq, cnt = _gather_counts(qc, idx)
    loss, perp = _losses(zf, zq, cnt.reshape(-1))
    return zq.reshape(z.shape), loss[0, 0], perp[0, 0]


# speedup vs baseline: 1.7378x; 1.7378x over previous
import functools

import jax
import jax.numpy as jnp
from jax import lax
from jax.experimental import pallas as pl
from jax.experimental.pallas import tpu as pltpu
from jax.experimental.pallas import tpu_sc as plsc

K = 8192
D = 64
N = 16384
BETA = 0.25

TM = 512
TK = 512


def _project_body(cb_ref, pw_ref, pb_ref, qc_ref, qcm2_ref, qcsq_ref):
    qc = lax.dot_general(
        cb_ref[...], pw_ref[...],
        dimension_numbers=(((1,), (1,)), ((), ())),
        preferred_element_type=jnp.float32,
        precision=lax.Precision.DEFAULT,
    ) + pb_ref[...]
    qc_ref[...] = qc
    qcm2_ref[...] = -2.0 * qc
    qcsq_ref[...] = jnp.sum(qc * qc, axis=1)


def _project(codebook, proj_w, proj_b):
    return pl.pallas_call(
        _project_body,
        out_shape=(
            jax.ShapeDtypeStruct((K, D), jnp.float32),
            jax.ShapeDtypeStruct((K, D), jnp.float32),
            jax.ShapeDtypeStruct((K,), jnp.float32),
        ),
    )(codebook, proj_w, proj_b.reshape(1, D))


_LANES = 128
_NSL = TK // _LANES


def _argmin_body(z_ref, qcm2_ref, qcsq_ref, out_ref, zsq_ref, rmin_ref, rcid_ref):
    j = pl.program_id(1)
    nj = pl.num_programs(1)
    zb = z_ref[...]

    @pl.when(j == 0)
    def _():
        zsq = jnp.sum(zb * zb, axis=1)
        zsq_ref[...] = jnp.broadcast_to(zsq[:, None], (TM, _LANES))
        rmin_ref[...] = jnp.full((TM, _LANES), jnp.inf, jnp.float32)
        rcid_ref[...] = jnp.zeros((TM, _LANES), jnp.int32)

    s2 = lax.dot_general(
        zb, qcm2_ref[...],
        dimension_numbers=(((1,), (1,)), ((), ())),
        preferred_element_type=jnp.float32,
        precision=lax.Precision.DEFAULT,
    )
    zs = zsq_ref[...]
    rm = rmin_ref[...]
    rc = rcid_ref[...]
    for k in range(_NSL):
        qk = jnp.broadcast_to(qcsq_ref[0, pl.ds(k, 1), :], (TM, _LANES))
        dk = (zs + qk) + s2[:, k * _LANES:(k + 1) * _LANES]
        better = dk < rm
        rm = jnp.where(better, dk, rm)
        rc = jnp.where(better, jnp.full((TM, _LANES), 1, jnp.int32) * (j * _NSL + k), rc)
    rmin_ref[...] = rm
    rcid_ref[...] = rc

    @pl.when(j == nj - 1)
    def _():
        m = jnp.min(rm, axis=1)
        lane = lax.broadcasted_iota(jnp.int32, (TM, _LANES), 1)
        glob = rc * _LANES + lane
        masked = jnp.where(rm == m[:, None], glob, jnp.int32(2**30))
        out_ref[...] = jnp.min(masked, axis=1)


def _argmin(zf, qcm2, qcsq2d):
    return pl.pallas_call(
        _argmin_body,
        grid=(N // TM, K // TK),
        in_specs=[
            pl.BlockSpec((TM, D), lambda i, j: (i, 0)),
            pl.BlockSpec((TK, D), lambda i, j: (j, 0)),
            pl.BlockSpec((1, _NSL, _LANES), lambda i, j: (j, 0, 0)),
        ],
        out_specs=pl.BlockSpec((TM,), lambda i, j: (i,)),
        out_shape=jax.ShapeDtypeStruct((N,), jnp.int32),
        scratch_shapes=[
            pltpu.VMEM((TM, _LANES), jnp.float32),
            pltpu.VMEM((TM, _LANES), jnp.float32),
            pltpu.VMEM((TM, _LANES), jnp.int32),
        ],
        compiler_params=pltpu.CompilerParams(
            dimension_semantics=("arbitrary", "arbitrary"),
        ),
    )(zf, qcm2, qcsq2d)


_SC_NC = 2
_SC_NS = 16
_BPW = N // (_SC_NC * _SC_NS)
_KPS = K // _SC_NS


def _sc_body(qc_hbm, idx_hbm, zq_hbm, cnt_hbm,
             idx_v, rows_v, stage_v, ones_v, cnt_sh, sem):
    c = lax.axis_index("c")
    s = lax.axis_index("s")
    wid = c * _SC_NS + s
    base = wid * _BPW

    pltpu.sync_copy(idx_hbm.at[pl.ds(base, _BPW)], idx_v)
    pltpu.async_copy(qc_hbm.at[idx_v], rows_v, sem).wait()
    pltpu.sync_copy(rows_v, zq_hbm.at[pl.ds(base, _BPW)])

    def fill(i, _):
        ones_v[pl.ds(i * 16, 16)] = jnp.full((16,), 1, jnp.int32)
        stage_v[pl.ds(i * 16, 16)] = jnp.full((16,), 0, jnp.int32)
        return 0

    lax.fori_loop(0, _BPW // 16, fill, 0)

    pltpu.sync_copy(stage_v, cnt_sh.at[pl.ds(s * _KPS, _KPS)])
    plsc.subcore_barrier()
    pltpu.sync_copy(ones_v, cnt_sh.at[idx_v], add=True)
    plsc.subcore_barrier()

    pltpu.sync_copy(cnt_sh.at[pl.ds(s * _KPS, _KPS)], stage_v)
    pltpu.sync_copy(stage_v, cnt_hbm.at[c, pl.ds(s * _KPS, _KPS)])


def _gather_counts(qc, idx):
    mesh = plsc.VectorSubcoreMesh(core_axis_name="c", subcore_axis_name="s")
    f = pl.kernel(
        _sc_body,
        out_type=(
            jax.ShapeDtypeStruct((N, D), jnp.float32),
            jax.ShapeDtypeStruct((_SC_NC, K), jnp.int32),
        ),
        mesh=mesh,
        scratch_types=[
            pltpu.VMEM((_BPW,), jnp.int32),
            pltpu.VMEM((_BPW, D), jnp.float32),
            pltpu.VMEM((_KPS,), jnp.int32),
            pltpu.VMEM((_BPW,), jnp.int32),
            pltpu.VMEM_SHARED((K,), jnp.int32),
            pltpu.SemaphoreType.DMA,
        ],
        compiler_params=pltpu.CompilerParams(use_tc_tiling_on_sc=False),
    )
    return f(qc, idx)


def _loss_body(zf_ref, zq_ref, cnt_ref, loss_ref, perp_ref):
    diff = zq_ref[...] - zf_ref[...]
    sq = jnp.sum(diff * diff)
    loss_ref[0, 0] = (1.0 + BETA) * sq / jnp.float32(N * D)
    counts = cnt_ref[0:K] + cnt_ref[K:2 * K]
    e = counts.astype(jnp.float32) * jnp.float32(1.0 / N)
    ent = jnp.sum(e * jnp.log(e + 1e-8))
    perp_ref[0, 0] = jnp.exp(-ent)


def _losses(zf, zq, cnt):
    return pl.pallas_call(
        _loss_body,
        in_specs=[
            pl.BlockSpec(memory_space=pltpu.VMEM),
            pl.BlockSpec(memory_space=pltpu.VMEM),
            pl.BlockSpec(memory_space=pltpu.VMEM),
        ],
        out_specs=(
            pl.BlockSpec(memory_space=pltpu.SMEM),
            pl.BlockSpec(memory_space=pltpu.SMEM),
        ),
        out_shape=(
            jax.ShapeDtypeStruct((1, 1), jnp.float32),
            jax.ShapeDtypeStruct((1, 1), jnp.float32),
        ),
    )(zf, zq, cnt)


def kernel(z, codebook, proj_w, proj_b):
    zf = z.reshape(-1, D)
    qc, qcm2, qcsq = _project(codebook, proj_w, proj_b)
    idx = _argmin(zf, qcm2, qcsq.reshape(K // TK, _NSL, _LANES))
    zq, cnt = _gather_counts(qc, idx)
    loss, perp = _losses(zf, zq, cnt.reshape(-1))
    return zq.reshape(z.shape), loss[0, 0], perp[0, 0]


# speedup vs baseline: 2.3980x; 1.3799x over previous
import functools

import jax
import jax.numpy as jnp
from jax import lax
from jax.experimental import pallas as pl
from jax.experimental.pallas import tpu as pltpu
from jax.experimental.pallas import tpu_sc as plsc

K = 8192
D = 64
N = 16384
BETA = 0.25

TM = 512
TK = 1024


def _project_body(cb_ref, pw_ref, pb_ref, qc_ref, qcm2_ref, qcsq_ref):
    qc = lax.dot_general(
        cb_ref[...], pw_ref[...],
        dimension_numbers=(((1,), (1,)), ((), ())),
        preferred_element_type=jnp.float32,
        precision=lax.Precision.DEFAULT,
    ) + pb_ref[...]
    qc_ref[...] = qc
    qcm2_ref[...] = -2.0 * qc
    qcsq_ref[...] = jnp.sum(qc * qc, axis=1)


def _project(codebook, proj_w, proj_b):
    return pl.pallas_call(
        _project_body,
        out_shape=(
            jax.ShapeDtypeStruct((K, D), jnp.float32),
            jax.ShapeDtypeStruct((K, D), jnp.float32),
            jax.ShapeDtypeStruct((K,), jnp.float32),
        ),
    )(codebook, proj_w, proj_b.reshape(1, D))


_LANES = 128
_NSL = TK // _LANES


def _scan_body(z_ref, qcm2_ref, qcsq_ref, rmin_ref, rcid_ref, zsq_ref):
    j = pl.program_id(1)
    zb = z_ref[...]

    @pl.when(j == 0)
    def _():
        zsq = jnp.sum(zb * zb, axis=1)
        zsq_ref[...] = jnp.broadcast_to(zsq[:, None], (TM, _LANES))
        rmin_ref[...] = jnp.full((TM, _LANES), jnp.inf, jnp.float32)
        rcid_ref[...] = jnp.zeros((TM, _LANES), jnp.int32)

    s2 = lax.dot_general(
        zb, qcm2_ref[...],
        dimension_numbers=(((1,), (1,)), ((), ())),
        preferred_element_type=jnp.float32,
        precision=lax.Precision.DEFAULT,
    )
    zs = zsq_ref[...]
    ds = []
    for k in range(_NSL):
        qk = jnp.broadcast_to(qcsq_ref[0, pl.ds(k, 1), :], (TM, _LANES))
        ds.append(((zs + qk) + s2[:, k * _LANES:(k + 1) * _LANES], k))
    while len(ds) > 1:
        nxt = []
        for a in range(0, len(ds), 2):
            (d0, c0), (d1, c1) = ds[a], ds[a + 1]
            lt = d1 < d0
            if isinstance(c0, int):
                cw = jnp.where(lt, jnp.int32(c1), jnp.int32(c0))
            else:
                cw = jnp.where(lt, c1, c0)
            nxt.append((jnp.minimum(d0, d1), cw))
        ds = nxt
    dw, cw = ds[0]
    rm = rmin_ref[...]
    lt = dw < rm
    rmin_ref[...] = jnp.minimum(rm, dw)
    rcid_ref[...] = jnp.where(lt, cw + j * _NSL, rcid_ref[...])


def _reduce_body(rmin_ref, rcid_ref, out_ref):
    rm = rmin_ref[...]
    m = jnp.min(rm, axis=1)
    lane = lax.broadcasted_iota(jnp.int32, (TM, _LANES), 1)
    glob = rcid_ref[...] * _LANES + lane
    masked = jnp.where(rm == m[:, None], glob, jnp.int32(2**30))
    out_ref[...] = jnp.min(masked, axis=1)


def _argmin(zf, qcm2, qcsq2d):
    rm, rc = pl.pallas_call(
        _scan_body,
        grid=(N // TM, K // TK),
        in_specs=[
            pl.BlockSpec((TM, D), lambda i, j: (i, 0)),
            pl.BlockSpec((TK, D), lambda i, j: (j, 0)),
            pl.BlockSpec((1, _NSL, _LANES), lambda i, j: (j, 0, 0)),
        ],
        out_specs=(
            pl.BlockSpec((TM, _LANES), lambda i, j: (i, 0)),
            pl.BlockSpec((TM, _LANES), lambda i, j: (i, 0)),
        ),
        out_shape=(
            jax.ShapeDtypeStruct((N, _LANES), jnp.float32),
            jax.ShapeDtypeStruct((N, _LANES), jnp.int32),
        ),
        scratch_shapes=[
            pltpu.VMEM((TM, _LANES), jnp.float32),
        ],
        compiler_params=pltpu.CompilerParams(
            dimension_semantics=("arbitrary", "arbitrary"),
        ),
    )(zf, qcm2, qcsq2d)
    return pl.pallas_call(
        _reduce_body,
        grid=(N // TM,),
        in_specs=[
            pl.BlockSpec((TM, _LANES), lambda i: (i, 0)),
            pl.BlockSpec((TM, _LANES), lambda i: (i, 0)),
        ],
        out_specs=pl.BlockSpec((TM,), lambda i: (i,)),
        out_shape=jax.ShapeDtypeStruct((N,), jnp.int32),
    )(rm, rc)


_SC_NC = 2
_SC_NS = 16
_BPW = N // (_SC_NC * _SC_NS)
_KPS = K // _SC_NS


def _sc_body(qc_hbm, idx_hbm, zq_hbm, cnt_hbm,
             idx_v, rows_v, stage_v, ones_v, cnt_sh, sem):
    c = lax.axis_index("c")
    s = lax.axis_index("s")
    wid = c * _SC_NS + s
    base = wid * _BPW

    pltpu.sync_copy(idx_hbm.at[pl.ds(base, _BPW)], idx_v)
    pltpu.async_copy(qc_hbm.at[idx_v], rows_v, sem).wait()
    pltpu.sync_copy(rows_v, zq_hbm.at[pl.ds(base, _BPW)])

    def fill(i, _):
        ones_v[pl.ds(i * 16, 16)] = jnp.full((16,), 1, jnp.int32)
        stage_v[pl.ds(i * 16, 16)] = jnp.full((16,), 0, jnp.int32)
        return 0

    lax.fori_loop(0, _BPW // 16, fill, 0)

    pltpu.sync_copy(stage_v, cnt_sh.at[pl.ds(s * _KPS, _KPS)])
    plsc.subcore_barrier()
    pltpu.sync_copy(ones_v, cnt_sh.at[idx_v], add=True)
    plsc.subcore_barrier()

    pltpu.sync_copy(cnt_sh.at[pl.ds(s * _KPS, _KPS)], stage_v)
    pltpu.sync_copy(stage_v, cnt_hbm.at[c, pl.ds(s * _KPS, _KPS)])


def _gather_counts(qc, idx):
    mesh = plsc.VectorSubcoreMesh(core_axis_name="c", subcore_axis_name="s")
    f = pl.kernel(
        _sc_body,
        out_type=(
            jax.ShapeDtypeStruct((N, D), jnp.float32),
            jax.ShapeDtypeStruct((_SC_NC, K), jnp.int32),
        ),
        mesh=mesh,
        scratch_types=[
            pltpu.VMEM((_BPW,), jnp.int32),
            pltpu.VMEM((_BPW, D), jnp.float32),
            pltpu.VMEM((_KPS,), jnp.int32),
            pltpu.VMEM((_BPW,), jnp.int32),
            pltpu.VMEM_SHARED((K,), jnp.int32),
            pltpu.SemaphoreType.DMA,
        ],
        compiler_params=pltpu.CompilerParams(use_tc_tiling_on_sc=False),
    )
    return f(qc, idx)


def _loss_body(zf_ref, zq_ref, cnt_ref, loss_ref, perp_ref):
    diff = zq_ref[...] - zf_ref[...]
    sq = jnp.sum(diff * diff)
    loss_ref[0, 0] = (1.0 + BETA) * sq / jnp.float32(N * D)
    counts = cnt_ref[0:K] + cnt_ref[K:2 * K]
    e = counts.astype(jnp.float32) * jnp.float32(1.0 / N)
    ent = jnp.sum(e * jnp.log(e + 1e-8))
    perp_ref[0, 0] = jnp.exp(-ent)


def _losses(zf, zq, cnt):
    return pl.pallas_call(
        _loss_body,
        in_specs=[
            pl.BlockSpec(memory_space=pltpu.VMEM),
            pl.BlockSpec(memory_space=pltpu.VMEM),
            pl.BlockSpec(memory_space=pltpu.VMEM),
        ],
        out_specs=(
            pl.BlockSpec(memory_space=pltpu.SMEM),
            pl.BlockSpec(memory_space=pltpu.SMEM),
        ),
        out_shape=(
            jax.ShapeDtypeStruct((1, 1), jnp.float32),
            jax.ShapeDtypeStruct((1, 1), jnp.float32),
        ),
    )(zf, zq, cnt)


def kernel(z, codebook, proj_w, proj_b):
    zf = z.reshape(-1, D)
    qc, qcm2, qcsq = _project(codebook, proj_w, proj_b)
    idx = _argmin(zf, qcm2, qcsq.reshape(K // TK, _NSL, _LANES))
    zq, cnt = _gather_counts(qc, idx)
    loss, perp = _losses(zf, zq, cnt.reshape(-1))
    return zq.reshape(z.shape), loss[0, 0], perp[0, 0]
